# baseline (speedup 1.0000x reference)
import functools

import jax
import jax.numpy as jnp
from jax import lax
from jax.experimental import pallas as pl
from jax.experimental.pallas import tpu as pltpu
from jax.experimental.pallas import tpu_sc as plsc

_N = 10000
_NPAD = 10240
_E = 320000
_D = 128
_DE = 16
_G = 64
_NS = 16
_EW = _E // _NS
_C = 40
_NCHUNK = _EW // _C
_ROWS_PER = _NPAD // _NS

_BLK = 2048
_NBLK = _NPAD // _BLK
_EBLK = 8000
_NEBLK = _E // _EBLK


def _proj_body(x_ref, wq_ref, wkv_ref, wr_ref, q_ref, kv_ref, xr_ref):
    xv = x_ref[...]
    q_ref[...] = jnp.dot(xv, wq_ref[...], preferred_element_type=jnp.float32)
    kv_ref[...] = jnp.dot(xv, wkv_ref[...], preferred_element_type=jnp.float32)
    xr_ref[...] = jnp.dot(xv, wr_ref[...], preferred_element_type=jnp.float32)


def _proj(x_pad, wq, wkv, wroot):
    return pl.pallas_call(
        _proj_body,
        grid=(_NBLK,),
        in_specs=[
            pl.BlockSpec((_BLK, _D), lambda i: (i, 0)),
            pl.BlockSpec((_D, _D), lambda i: (0, 0)),
            pl.BlockSpec((_D, 2 * _D), lambda i: (0, 0)),
            pl.BlockSpec((_D, _D), lambda i: (0, 0)),
        ],
        out_specs=[
            pl.BlockSpec((_BLK, _D), lambda i: (i, 0)),
            pl.BlockSpec((_BLK, 2 * _D), lambda i: (i, 0)),
            pl.BlockSpec((_BLK, _D), lambda i: (i, 0)),
        ],
        out_shape=[
            jax.ShapeDtypeStruct((_NPAD, _D), jnp.float32),
            jax.ShapeDtypeStruct((_NPAD, 2 * _D), jnp.float32),
            jax.ShapeDtypeStruct((_NPAD, _D), jnp.float32),
        ],
    )(x_pad, wq, wkv, wroot)


def _eproj_body(ea_ref, we_ref, e_ref):
    e_ref[...] = jnp.dot(ea_ref[...], we_ref[...],
                         preferred_element_type=jnp.float32)


def _eproj(edge_attr, we):
    return pl.pallas_call(
        _eproj_body,
        grid=(_NEBLK,),
        in_specs=[
            pl.BlockSpec((_EBLK, _DE), lambda i: (i, 0)),
            pl.BlockSpec((_DE, _D), lambda i: (0, 0)),
        ],
        out_specs=pl.BlockSpec((_EBLK, _D), lambda i: (i, 0)),
        out_shape=jax.ShapeDtypeStruct((_E, _D), jnp.float32),
    )(edge_attr, we)


def _sc_edge_body(src_hbm, dst_hbm, q_hbm, kv_hbm, e_hbm, out_hbm,
                  srcb0, srcb1, srcb2, srcb3,
                  dstb0, dstb1, dstb2, dstb3, sidx,
                  qv0, qv1, kvv0, kvv1, ev0, ev1,
                  msgv, acc,
                  isem0, isem1, isem2, isem3, gsem0, gsem1, ssem):
    sid = lax.axis_index("s")

    srcb = (srcb0, srcb1, srcb2, srcb3)
    dstb = (dstb0, dstb1, dstb2, dstb3)
    qv = (qv0, qv1)
    kvv = (kvv0, kvv1)
    ev = (ev0, ev1)
    isem = (isem0, isem1, isem2, isem3)
    gsem = (gsem0, gsem1)

    zero = jnp.zeros((16,), jnp.float32)

    def _zr(r, carry):
        for cc in range(_D // 16):
            msgv[r, pl.ds(cc * 16, 16)] = zero
        return carry

    lax.fori_loop(0, _C, _zr, 0)
    for t in range(_ROWS_PER // _C):
        pltpu.sync_copy(msgv, acc.at[pl.ds(sid * _ROWS_PER + t * _C, _C)])
    plsc.subcore_barrier()

    def _idx_start(j, ip):
        eb = pl.multiple_of(sid * _EW + j * _C, 8)
        pltpu.make_async_copy(src_hbm.at[pl.ds(eb, _C)], srcb[ip],
                              isem[ip]).start()
        pltpu.make_async_copy(dst_hbm.at[pl.ds(eb, _C)], dstb[ip],
                              isem[ip]).start()

    def _idx_wait(ip):
        pltpu.make_async_copy(src_hbm.at[pl.ds(0, _C)], srcb[ip],
                              isem[ip]).wait()
        pltpu.make_async_copy(dst_hbm.at[pl.ds(0, _C)], dstb[ip],
                              isem[ip]).wait()

    def _gather_start(j, ip, p):
        eb = pl.multiple_of(sid * _EW + j * _C, 8)
        pltpu.make_async_copy(q_hbm.at[dstb[ip]], qv[p], gsem[p]).start()
        pltpu.make_async_copy(kv_hbm.at[srcb[ip]], kvv[p], gsem[p]).start()
        pltpu.make_async_copy(e_hbm.at[pl.ds(eb, _C)], ev[p], gsem[p]).start()

    def _gather_wait(ip, p):
        pltpu.make_async_copy(q_hbm.at[dstb[ip]], qv[p], gsem[p]).wait()
        pltpu.make_async_copy(kv_hbm.at[srcb[ip]], kvv[p], gsem[p]).wait()
        pltpu.make_async_copy(e_hbm.at[pl.ds(0, _C)], ev[p], gsem[p]).wait()

    def _unit(j, u, m):
        ip = u % 4
        p = u % 2
        _gather_wait(ip, p)

        @pl.when(j > 0)
        def _():
            pltpu.make_async_copy(msgv, acc.at[sidx], ssem).wait()

        for off in (0, 16, 24):
            sidx[pl.ds(off, 16)] = dstb[ip][pl.ds(off, 16)]

        @plsc.parallel_loop(0, _C, unroll=8)
        def _row(r):
            for cc in range(_D // 16):
                sl = pl.ds(cc * 16, 16)
                z = qv[p][r, sl] + kvv[p][r, sl] + ev[p][r, sl]
                g = 1.0 / (1.0 + jnp.exp(-z))
                msgv[r, sl] = g * kvv[p][r, pl.ds(_D + cc * 16, 16)]
        pltpu.make_async_copy(msgv, acc.at[sidx], ssem).start(add=True)

        @pl.when(j + 4 < _NCHUNK)
        def _():
            _idx_start(j + 4, ip)

        @pl.when(j + 2 < _NCHUNK)
        def _():
            _idx_wait((u + 2) % 4)
            _gather_start(j + 2, (u + 2) % 4, p)

    for u in range(4):
        _idx_start(u, u)
    _idx_wait(0)
    _gather_start(0, 0, 0)
    _idx_wait(1)
    _gather_start(1, 1, 1)

    def _quad(m, carry):
        for u in range(4):
            _unit(4 * m + u, u, m)
        return carry

    lax.fori_loop(0, _NCHUNK // 4, _quad, 0)

    pltpu.make_async_copy(msgv, acc.at[sidx], ssem).wait()
    plsc.subcore_barrier()
    pltpu.sync_copy(acc.at[pl.ds(sid * _ROWS_PER, _ROWS_PER)],
                    out_hbm.at[pl.ds(sid * _ROWS_PER, _ROWS_PER)])


_sc_edge = functools.partial(
    pl.kernel,
    mesh=plsc.VectorSubcoreMesh(core_axis_name="c", subcore_axis_name="s",
                                num_cores=1),
    out_type=jax.ShapeDtypeStruct((_NPAD, _D), jnp.float32),
    scratch_types=(
        [pltpu.VMEM((_C,), jnp.int32)] * 9
        + [pltpu.VMEM((_C, _D), jnp.float32)] * 2
        + [pltpu.VMEM((_C, 2 * _D), jnp.float32)] * 2
        + [pltpu.VMEM((_C, _D), jnp.float32)] * 2
        + [pltpu.VMEM((_C, _D), jnp.float32)]
        + [pltpu.VMEM_SHARED((_NPAD, _D), jnp.float32)]
        + [pltpu.SemaphoreType.DMA] * 7
    ),
)(_sc_edge_body)


def _fin_body(xr_ref, p_ref, batch_ref, b_ref, out_ref, sums, counts):
    i = pl.program_id(0)

    @pl.when(i == 0)
    def _():
        sums[...] = jnp.zeros_like(sums)
        counts[...] = jnp.zeros_like(counts)

    o = jnp.maximum(xr_ref[...] + p_ref[...] + b_ref[...], 0.0)
    bv = batch_ref[0]
    gid = lax.broadcasted_iota(jnp.int32, (_G, _BLK), 0)
    rows = lax.broadcasted_iota(jnp.int32, (_G, _BLK), 1) + i * _BLK
    onehot = jnp.where((bv == gid) & (rows < _N), 1.0, 0.0)
    sums[...] += jnp.dot(onehot, o, preferred_element_type=jnp.float32)
    counts[...] += jnp.dot(onehot, jnp.ones((_BLK, _D), jnp.float32),
                           preferred_element_type=jnp.float32)

    @pl.when(i == _NBLK - 1)
    def _():
        out_ref[...] = sums[...] / jnp.maximum(counts[...], 1.0)


def _finalize(xr, p, batch3, b2):
    return pl.pallas_call(
        _fin_body,
        grid=(_NBLK,),
        in_specs=[
            pl.BlockSpec((_BLK, _D), lambda i: (i, 0)),
            pl.BlockSpec((_BLK, _D), lambda i: (i, 0)),
            pl.BlockSpec((1, 1, _BLK), lambda i: (i, 0, 0)),
            pl.BlockSpec((1, _D), lambda i: (0, 0)),
        ],
        out_specs=pl.BlockSpec((_G, _D), lambda i: (0, 0)),
        out_shape=jax.ShapeDtypeStruct((_G, _D), jnp.float32),
        scratch_shapes=[
            pltpu.VMEM((_G, _D), jnp.float32),
            pltpu.VMEM((_G, _D), jnp.float32),
        ],
    )(xr, p, batch3, b2)


def kernel(x, edge_index, edge_attr, batch, Wq, Wk, Wv, We, Wroot, b):
    src = edge_index[0]
    dst = edge_index[1]
    x_pad = jnp.pad(x, ((0, _NPAD - _N), (0, 0)))
    wkv = jnp.concatenate([Wk, Wv], axis=1)
    q, kv, xr = _proj(x_pad, Wq, wkv, Wroot)
    e = _eproj(edge_attr, We)
    p = _sc_edge(src, dst, q, kv, e)
    batch3 = jnp.pad(batch, (0, _NPAD - _N)).reshape(_NBLK, 1, _BLK)
    pooled = _finalize(xr, p, batch3, b.reshape(1, _D))
    return pooled


# speedup vs baseline: 1.0024x; 1.0024x over previous
import functools

import jax
import jax.numpy as jnp
from jax import lax
from jax.experimental import pallas as pl
from jax.experimental.pallas import tpu as pltpu
from jax.experimental.pallas import tpu_sc as plsc

_N = 10000
_NPAD = 10240
_E = 320000
_D = 128
_DE = 16
_G = 64
_NS = 16
_EW = _E // _NS
_C = 40
_NCHUNK = _EW // _C
_ROWS_PER = _NPAD // _NS

_BLK = 2048
_NBLK = _NPAD // _BLK
_EBLK = 8000
_NEBLK = _E // _EBLK


def _proj_body(x_ref, wq_ref, wkv_ref, wr_ref, q_ref, kv_ref, xr_ref):
    xv = x_ref[...]
    q_ref[...] = jnp.dot(xv, wq_ref[...], preferred_element_type=jnp.float32)
    kv_ref[...] = jnp.dot(xv, wkv_ref[...], preferred_element_type=jnp.float32)
    xr_ref[...] = jnp.dot(xv, wr_ref[...], preferred_element_type=jnp.float32)


def _proj(x_pad, wq, wkv, wroot):
    return pl.pallas_call(
        _proj_body,
        grid=(_NBLK,),
        in_specs=[
            pl.BlockSpec((_BLK, _D), lambda i: (i, 0)),
            pl.BlockSpec((_D, _D), lambda i: (0, 0)),
            pl.BlockSpec((_D, 2 * _D), lambda i: (0, 0)),
            pl.BlockSpec((_D, _D), lambda i: (0, 0)),
        ],
        out_specs=[
            pl.BlockSpec((_BLK, _D), lambda i: (i, 0)),
            pl.BlockSpec((_BLK, 2 * _D), lambda i: (i, 0)),
            pl.BlockSpec((_BLK, _D), lambda i: (i, 0)),
        ],
        out_shape=[
            jax.ShapeDtypeStruct((_NPAD, _D), jnp.float32),
            jax.ShapeDtypeStruct((_NPAD, 2 * _D), jnp.float32),
            jax.ShapeDtypeStruct((_NPAD, _D), jnp.float32),
        ],
    )(x_pad, wq, wkv, wroot)


def _eproj_body(ea_ref, we_ref, e_ref):
    e_ref[...] = jnp.dot(ea_ref[...], we_ref[...],
                         preferred_element_type=jnp.float32)


def _eproj(edge_attr, we):
    return pl.pallas_call(
        _eproj_body,
        grid=(_NEBLK,),
        in_specs=[
            pl.BlockSpec((_EBLK, _DE), lambda i: (i, 0)),
            pl.BlockSpec((_DE, _D), lambda i: (0, 0)),
        ],
        out_specs=pl.BlockSpec((_EBLK, _D), lambda i: (i, 0)),
        out_shape=jax.ShapeDtypeStruct((_E, _D), jnp.float32),
    )(edge_attr, we)


def _sc_edge_body(src_hbm, dst_hbm, q_hbm, kv_hbm, e_hbm, out_hbm,
                  srcb0, srcb1, srcb2, srcb3,
                  dstb0, dstb1, dstb2, dstb3, sidx,
                  qv0, qv1, kvv0, kvv1, ev0, ev1,
                  msgv, acc,
                  isem0, isem1, isem2, isem3, gsem0, gsem1, ssem):
    sid = lax.axis_index("s")

    srcb = (srcb0, srcb1, srcb2, srcb3)
    dstb = (dstb0, dstb1, dstb2, dstb3)
    qv = (qv0, qv1)
    kvv = (kvv0, kvv1)
    ev = (ev0, ev1)
    isem = (isem0, isem1, isem2, isem3)
    gsem = (gsem0, gsem1)

    zero = jnp.zeros((16,), jnp.float32)

    def _zr(r, carry):
        for cc in range(_D // 16):
            msgv[r, pl.ds(cc * 16, 16)] = zero
        return carry

    lax.fori_loop(0, _C, _zr, 0)
    for t in range(_ROWS_PER // _C):
        pltpu.sync_copy(msgv, acc.at[pl.ds(sid * _ROWS_PER + t * _C, _C)])
    plsc.subcore_barrier()

    def _idx_start(j, ip):
        eb = pl.multiple_of(sid * _EW + j * _C, 8)
        pltpu.make_async_copy(src_hbm.at[pl.ds(eb, _C)], srcb[ip],
                              isem[ip]).start()
        pltpu.make_async_copy(dst_hbm.at[pl.ds(eb, _C)], dstb[ip],
                              isem[ip]).start()

    def _idx_wait(ip):
        pltpu.make_async_copy(src_hbm.at[pl.ds(0, _C)], srcb[ip],
                              isem[ip]).wait()
        pltpu.make_async_copy(dst_hbm.at[pl.ds(0, _C)], dstb[ip],
                              isem[ip]).wait()

    def _gather_start(j, ip, p):
        eb = pl.multiple_of(sid * _EW + j * _C, 8)
        pltpu.make_async_copy(q_hbm.at[dstb[ip]], qv[p], gsem[p]).start()
        pltpu.make_async_copy(kv_hbm.at[srcb[ip]], kvv[p], gsem[p]).start()
        pltpu.make_async_copy(e_hbm.at[pl.ds(eb, _C)], ev[p], gsem[p]).start()

    def _gather_wait(ip, p):
        pltpu.make_async_copy(q_hbm.at[dstb[ip]], qv[p], gsem[p]).wait()
        pltpu.make_async_copy(kv_hbm.at[srcb[ip]], kvv[p], gsem[p]).wait()
        pltpu.make_async_copy(e_hbm.at[pl.ds(0, _C)], ev[p], gsem[p]).wait()

    def _unit(j, u, m):
        ip = u % 4
        p = u % 2
        _gather_wait(ip, p)

        @pl.when(j > 0)
        def _():
            pltpu.make_async_copy(msgv, acc.at[sidx], ssem).wait()

        for off in (0, 16, 24):
            sidx[pl.ds(off, 16)] = dstb[ip][pl.ds(off, 16)]

        @plsc.parallel_loop(0, _C, unroll=4)
        def _row(r):
            for cc in range(_D // 16):
                sl = pl.ds(cc * 16, 16)
                z = qv[p][r, sl] + kvv[p][r, sl] + ev[p][r, sl]
                g = 1.0 / (1.0 + jnp.exp(-z))
                msgv[r, sl] = g * kvv[p][r, pl.ds(_D + cc * 16, 16)]
        pltpu.make_async_copy(msgv, acc.at[sidx], ssem).start(add=True)

        @pl.when(j + 4 < _NCHUNK)
        def _():
            _idx_start(j + 4, ip)

        @pl.when(j + 2 < _NCHUNK)
        def _():
            _idx_wait((u + 2) % 4)
            _gather_start(j + 2, (u + 2) % 4, p)

    for u in range(4):
        _idx_start(u, u)
    _idx_wait(0)
    _gather_start(0, 0, 0)
    _idx_wait(1)
    _gather_start(1, 1, 1)

    def _quad(m, carry):
        for u in range(4):
            _unit(4 * m + u, u, m)
        return carry

    lax.fori_loop(0, _NCHUNK // 4, _quad, 0)

    pltpu.make_async_copy(msgv, acc.at[sidx], ssem).wait()
    plsc.subcore_barrier()
    pltpu.sync_copy(acc.at[pl.ds(sid * _ROWS_PER, _ROWS_PER)],
                    out_hbm.at[pl.ds(sid * _ROWS_PER, _ROWS_PER)])


_sc_edge = functools.partial(
    pl.kernel,
    mesh=plsc.VectorSubcoreMesh(core_axis_name="c", subcore_axis_name="s",
                                num_cores=1),
    out_type=jax.ShapeDtypeStruct((_NPAD, _D), jnp.float32),
    scratch_types=(
        [pltpu.VMEM((_C,), jnp.int32)] * 9
        + [pltpu.VMEM((_C, _D), jnp.float32)] * 2
        + [pltpu.VMEM((_C, 2 * _D), jnp.float32)] * 2
        + [pltpu.VMEM((_C, _D), jnp.float32)] * 2
        + [pltpu.VMEM((_C, _D), jnp.float32)]
        + [pltpu.VMEM_SHARED((_NPAD, _D), jnp.float32)]
        + [pltpu.SemaphoreType.DMA] * 7
    ),
)(_sc_edge_body)


def _fin_body(xr_ref, p_ref, batch_ref, b_ref, out_ref, sums, counts):
    i = pl.program_id(0)

    @pl.when(i == 0)
    def _():
        sums[...] = jnp.zeros_like(sums)
        counts[...] = jnp.zeros_like(counts)

    o = jnp.maximum(xr_ref[...] + p_ref[...] + b_ref[...], 0.0)
    bv = batch_ref[0]
    gid = lax.broadcasted_iota(jnp.int32, (_G, _BLK), 0)
    rows = lax.broadcasted_iota(jnp.int32, (_G, _BLK), 1) + i * _BLK
    onehot = jnp.where((bv == gid) & (rows < _N), 1.0, 0.0)
    sums[...] += jnp.dot(onehot, o, preferred_element_type=jnp.float32)
    counts[...] += jnp.dot(onehot, jnp.ones((_BLK, _D), jnp.float32),
                           preferred_element_type=jnp.float32)

    @pl.when(i == _NBLK - 1)
    def _():
        out_ref[...] = sums[...] / jnp.maximum(counts[...], 1.0)


def _finalize(xr, p, batch3, b2):
    return pl.pallas_call(
        _fin_body,
        grid=(_NBLK,),
        in_specs=[
            pl.BlockSpec((_BLK, _D), lambda i: (i, 0)),
            pl.BlockSpec((_BLK, _D), lambda i: (i, 0)),
            pl.BlockSpec((1, 1, _BLK), lambda i: (i, 0, 0)),
            pl.BlockSpec((1, _D), lambda i: (0, 0)),
        ],
        out_specs=pl.BlockSpec((_G, _D), lambda i: (0, 0)),
        out_shape=jax.ShapeDtypeStruct((_G, _D), jnp.float32),
        scratch_shapes=[
            pltpu.VMEM((_G, _D), jnp.float32),
            pltpu.VMEM((_G, _D), jnp.float32),
        ],
    )(xr, p, batch3, b2)


def kernel(x, edge_index, edge_attr, batch, Wq, Wk, Wv, We, Wroot, b):
    src = edge_index[0]
    dst = edge_index[1]
    x_pad = jnp.pad(x, ((0, _NPAD - _N), (0, 0)))
    wkv = jnp.concatenate([Wk, Wv], axis=1)
    q, kv, xr = _proj(x_pad, Wq, wkv, Wroot)
    e = _eproj(edge_attr, We)
    p = _sc_edge(src, dst, q, kv, e)
    batch3 = jnp.pad(batch, (0, _NPAD - _N)).reshape(_NBLK, 1, _BLK)
    pooled = _finalize(xr, p, batch3, b.reshape(1, _D))
    return pooled


# speedup vs baseline: 1.0688x; 1.0662x over previous
import functools

import jax
import jax.numpy as jnp
from jax import lax
from jax.experimental import pallas as pl
from jax.experimental.pallas import tpu as pltpu
from jax.experimental.pallas import tpu_sc as plsc

_N = 10000
_NPAD = 10240
_E = 320000
_D = 128
_DE = 16
_G = 64
_NS = 16
_EW = _E // _NS
_C = 40
_NCHUNK = _EW // _C
_ROWS_PER = _NPAD // _NS

_BLK = 2048
_NBLK = _NPAD // _BLK
_EBLK = 8000
_NEBLK = _E // _EBLK


def _proj_body(x_ref, wq_ref, wkv_ref, wr_ref, q_ref, kv_ref, xr_ref):
    xv = x_ref[...]
    q_ref[...] = jnp.dot(xv, wq_ref[...], preferred_element_type=jnp.float32)
    kv_ref[...] = jnp.dot(xv, wkv_ref[...], preferred_element_type=jnp.float32)
    xr_ref[...] = jnp.dot(xv, wr_ref[...], preferred_element_type=jnp.float32)


def _proj(x_pad, wq, wkv, wroot):
    return pl.pallas_call(
        _proj_body,
        grid=(_NBLK,),
        in_specs=[
            pl.BlockSpec((_BLK, _D), lambda i: (i, 0)),
            pl.BlockSpec((_D, _D), lambda i: (0, 0)),
            pl.BlockSpec((_D, 2 * _D), lambda i: (0, 0)),
            pl.BlockSpec((_D, _D), lambda i: (0, 0)),
        ],
        out_specs=[
            pl.BlockSpec((_BLK, _D), lambda i: (i, 0)),
            pl.BlockSpec((_BLK, 2 * _D), lambda i: (i, 0)),
            pl.BlockSpec((_BLK, _D), lambda i: (i, 0)),
        ],
        out_shape=[
            jax.ShapeDtypeStruct((_NPAD, _D), jnp.float32),
            jax.ShapeDtypeStruct((_NPAD, 2 * _D), jnp.float32),
            jax.ShapeDtypeStruct((_NPAD, _D), jnp.float32),
        ],
    )(x_pad, wq, wkv, wroot)


def _eproj_body(ea_ref, we_ref, e_ref):
    e_ref[...] = jnp.dot(ea_ref[...], we_ref[...],
                         preferred_element_type=jnp.float32)


def _eproj(edge_attr, we):
    return pl.pallas_call(
        _eproj_body,
        grid=(_NEBLK,),
        in_specs=[
            pl.BlockSpec((_EBLK, _DE), lambda i: (i, 0)),
            pl.BlockSpec((_DE, _D), lambda i: (0, 0)),
        ],
        out_specs=pl.BlockSpec((_EBLK, _D), lambda i: (i, 0)),
        out_shape=jax.ShapeDtypeStruct((_E, _D), jnp.float32),
    )(edge_attr, we)


def _sc_edge_body(src_hbm, dst_hbm, q_hbm, kv_hbm, e_hbm, out_hbm,
                  srcb0, srcb1, srcb2, srcb3,
                  dstb0, dstb1, dstb2, dstb3, sidx,
                  qv0, qv1, kvv0, kvv1, ev0, ev1,
                  msgv, acc,
                  isem0, isem1, isem2, isem3, gsem0, gsem1, ssem):
    sid = lax.axis_index("s")

    srcb = (srcb0, srcb1, srcb2, srcb3)
    dstb = (dstb0, dstb1, dstb2, dstb3)
    qv = (qv0, qv1)
    kvv = (kvv0, kvv1)
    ev = (ev0, ev1)
    isem = (isem0, isem1, isem2, isem3)
    gsem = (gsem0, gsem1)

    def _idx_start(j, ip):
        eb = pl.multiple_of(sid * _EW + j * _C, 8)
        pltpu.make_async_copy(src_hbm.at[pl.ds(eb, _C)], srcb[ip],
                              isem[ip]).start()
        pltpu.make_async_copy(dst_hbm.at[pl.ds(eb, _C)], dstb[ip],
                              isem[ip]).start()

    def _idx_wait(ip):
        pltpu.make_async_copy(src_hbm.at[pl.ds(0, _C)], srcb[ip],
                              isem[ip]).wait()
        pltpu.make_async_copy(dst_hbm.at[pl.ds(0, _C)], dstb[ip],
                              isem[ip]).wait()

    def _gather_start(j, ip, p):
        eb = pl.multiple_of(sid * _EW + j * _C, 8)
        pltpu.make_async_copy(q_hbm.at[dstb[ip]], qv[p], gsem[p]).start()
        pltpu.make_async_copy(kv_hbm.at[srcb[ip]], kvv[p], gsem[p]).start()
        pltpu.make_async_copy(e_hbm.at[pl.ds(eb, _C)], ev[p], gsem[p]).start()

    def _gather_wait(ip, p):
        pltpu.make_async_copy(q_hbm.at[dstb[ip]], qv[p], gsem[p]).wait()
        pltpu.make_async_copy(kv_hbm.at[srcb[ip]], kvv[p], gsem[p]).wait()
        pltpu.make_async_copy(e_hbm.at[pl.ds(0, _C)], ev[p], gsem[p]).wait()

    def _unit(j, u, m):
        ip = u % 4
        p = u % 2
        _gather_wait(ip, p)

        @pl.when(j > 0)
        def _():
            pltpu.make_async_copy(msgv, acc.at[sidx], ssem).wait()

        for off in (0, 16, 24):
            sidx[pl.ds(off, 16)] = dstb[ip][pl.ds(off, 16)]

        @plsc.parallel_loop(0, _C, unroll=4)
        def _row(r):
            for cc in range(_D // 16):
                sl = pl.ds(cc * 16, 16)
                z = qv[p][r, sl] + kvv[p][r, sl] + ev[p][r, sl]
                g = 1.0 / (1.0 + jnp.exp(-z))
                msgv[r, sl] = g * kvv[p][r, pl.ds(_D + cc * 16, 16)]
        pltpu.make_async_copy(msgv, acc.at[sidx], ssem).start(add=True)

        @pl.when(j + 4 < _NCHUNK)
        def _():
            _idx_start(j + 4, ip)

        @pl.when(j + 2 < _NCHUNK)
        def _():
            _idx_wait((u + 2) % 4)
            _gather_start(j + 2, (u + 2) % 4, p)

    for u in range(4):
        _idx_start(u, u)
    _idx_wait(0)
    _gather_start(0, 0, 0)
    _idx_wait(1)
    _gather_start(1, 1, 1)

    zero = jnp.zeros((16,), jnp.float32)

    def _zr(r, carry):
        for cc in range(_D // 16):
            msgv[r, pl.ds(cc * 16, 16)] = zero
        return carry

    lax.fori_loop(0, _C, _zr, 0)
    for t in range(_ROWS_PER // _C):
        pltpu.sync_copy(msgv, acc.at[pl.ds(sid * _ROWS_PER + t * _C, _C)])
    plsc.subcore_barrier()

    def _quad(m, carry):
        for u in range(4):
            _unit(4 * m + u, u, m)
        return carry

    lax.fori_loop(0, _NCHUNK // 4, _quad, 0)

    pltpu.make_async_copy(msgv, acc.at[sidx], ssem).wait()
    plsc.subcore_barrier()
    pltpu.sync_copy(acc.at[pl.ds(sid * _ROWS_PER, _ROWS_PER)],
                    out_hbm.at[pl.ds(sid * _ROWS_PER, _ROWS_PER)])


_sc_edge = functools.partial(
    pl.kernel,
    mesh=plsc.VectorSubcoreMesh(core_axis_name="c", subcore_axis_name="s",
                                num_cores=1),
    out_type=jax.ShapeDtypeStruct((_NPAD, _D), jnp.float32),
    scratch_types=(
        [pltpu.VMEM((_C,), jnp.int32)] * 9
        + [pltpu.VMEM((_C, _D), jnp.float32)] * 2
        + [pltpu.VMEM((_C, 2 * _D), jnp.float32)] * 2
        + [pltpu.VMEM((_C, _D), jnp.float32)] * 2
        + [pltpu.VMEM((_C, _D), jnp.float32)]
        + [pltpu.VMEM_SHARED((_NPAD, _D), jnp.float32)]
        + [pltpu.SemaphoreType.DMA] * 7
    ),
)(_sc_edge_body)


def _fin_body(xr_ref, p_ref, batch_ref, b_ref, out_ref, sums, counts):
    i = pl.program_id(0)

    @pl.when(i == 0)
    def _():
        sums[...] = jnp.zeros_like(sums)
        counts[...] = jnp.zeros_like(counts)

    o = jnp.maximum(xr_ref[...] + p_ref[...] + b_ref[...], 0.0)
    bv = batch_ref[0]
    gid = lax.broadcasted_iota(jnp.int32, (_G, _BLK), 0)
    rows = lax.broadcasted_iota(jnp.int32, (_G, _BLK), 1) + i * _BLK
    onehot = jnp.where((bv == gid) & (rows < _N), 1.0, 0.0)
    sums[...] += jnp.dot(onehot, o, preferred_element_type=jnp.float32)
    counts[...] += jnp.dot(onehot, jnp.ones((_BLK, _D), jnp.float32),
                           preferred_element_type=jnp.float32)

    @pl.when(i == _NBLK - 1)
    def _():
        out_ref[...] = sums[...] / jnp.maximum(counts[...], 1.0)


def _finalize(xr, p, batch3, b2):
    return pl.pallas_call(
        _fin_body,
        grid=(_NBLK,),
        in_specs=[
            pl.BlockSpec((_BLK, _D), lambda i: (i, 0)),
            pl.BlockSpec((_BLK, _D), lambda i: (i, 0)),
            pl.BlockSpec((1, 1, _BLK), lambda i: (i, 0, 0)),
            pl.BlockSpec((1, _D), lambda i: (0, 0)),
        ],
        out_specs=pl.BlockSpec((_G, _D), lambda i: (0, 0)),
        out_shape=jax.ShapeDtypeStruct((_G, _D), jnp.float32),
        scratch_shapes=[
            pltpu.VMEM((_G, _D), jnp.float32),
            pltpu.VMEM((_G, _D), jnp.float32),
        ],
    )(xr, p, batch3, b2)


def kernel(x, edge_index, edge_attr, batch, Wq, Wk, Wv, We, Wroot, b):
    src = edge_index[0]
    dst = edge_index[1]
    x_pad = jnp.pad(x, ((0, _NPAD - _N), (0, 0)))
    wkv = jnp.concatenate([Wk, Wv], axis=1)
    q, kv, xr = _proj(x_pad, Wq, wkv, Wroot)
    e = _eproj(edge_attr, We)
    p = _sc_edge(src, dst, q, kv, e)
    batch3 = jnp.pad(batch, (0, _NPAD - _N)).reshape(_NBLK, 1, _BLK)
    pooled = _finalize(xr, p, batch3, b.reshape(1, _D))
    return pooled
